# baseline (speedup 1.0000x reference)
import functools

import jax
import jax.numpy as jnp
from jax import lax
from jax.experimental import pallas as pl
from jax.experimental.pallas import tpu as pltpu
from jax.experimental.pallas import tpu_sc as plsc

_FWD_WEIGHT = 1.0

_NW = 32
_CH = 32768


def _sc_body(x_hbm, g_hbm, m_hbm, out_hbm, xb, gb, mb, accv):
    wid = lax.axis_index("s") * 2 + lax.axis_index("c")
    n = x_hbm.shape[0]
    per_w = n // _NW
    base = wid * per_w
    n_chunks = per_w // _CH

    lane = lax.iota(jnp.int32, 16)
    lq = lane >> 2
    sel = [1 - jnp.minimum(jnp.abs(lq - k), 1) for k in range(4)]


    def chunk_body(ci, acc):
        xoff = pl.multiple_of(base + ci * _CH, 8)
        moff = pl.multiple_of((base + ci * _CH) // 8, 8)
        pltpu.sync_copy(x_hbm.at[pl.ds(xoff, _CH)], xb)
        pltpu.sync_copy(g_hbm.at[pl.ds(xoff, _CH)], gb)
        pltpu.sync_copy(m_hbm.at[pl.ds(moff, _CH // 8)], mb.at[pl.ds(0, _CH // 8)])

        def pair_body(p, acc2):
            pr = p // 8
            off = p % 8
            c0 = pr * 256 + 16 * off
            xv0 = xb[pl.ds(c0, 16)]
            gv0 = gb[pl.ds(c0, 16)]
            xv1 = xb[pl.ds(c0 + 128, 16)]
            gv1 = gb[pl.ds(c0 + 128, 16)]
            mwv = mb[pl.ds(pr * 32 + 4 * off, 16)]
            lq = lane >> 2
            mg = (
                mwv[0] * sel[0]
                + mwv[1] * sel[1]
                + mwv[2] * sel[2]
                + mwv[3] * sel[3]
            )
            mbit = (mg >> ((lane & 3) << 3)) & 1
            mf = mbit.astype(jnp.float32)
            d = jnp.abs(xv0 - gv0) + jnp.abs(xv1 - gv1)
            return acc2 + d * mf

        return lax.fori_loop(0, _CH // 32, pair_body, acc)

    acc = lax.fori_loop(0, n_chunks, chunk_body, jnp.zeros((16,), jnp.float32))
    accv[...] = acc
    pltpu.sync_copy(accv, out_hbm.at[wid])


def kernel(uv_points, uv_gt, object_mask):
    B, H, W, C = uv_points.shape
    n = B * H * W * C

    def as_flat(a):
        return (
            a.reshape(B, H, W // 128, 128, C)
            .transpose(0, 1, 2, 4, 3)
            .reshape(n)
        )

    x = as_flat(uv_points)
    g = as_flat(uv_gt)
    m32 = lax.bitcast_convert_type(
        object_mask.view(jnp.uint8).reshape(n // 8, 4), jnp.int32
    )

    mesh = plsc.VectorSubcoreMesh(core_axis_name="c", subcore_axis_name="s")
    run = functools.partial(
        pl.kernel,
        mesh=mesh,
        out_type=jax.ShapeDtypeStruct((_NW, 16), jnp.float32),
        scratch_types=[
            pltpu.VMEM((_CH,), jnp.float32),
            pltpu.VMEM((_CH,), jnp.float32),
            pltpu.VMEM((_CH // 8 + 16,), jnp.int32),
            pltpu.VMEM((16,), jnp.float32),
        ],
    )(_sc_body)
    out = run(x, g, m32)

    uv_loss = jnp.sum(out) / float(B * H)
    return (_FWD_WEIGHT * uv_loss, uv_loss)


# speedup vs baseline: 24.6911x; 24.6911x over previous
import jax
import jax.numpy as jnp
from jax.experimental import pallas as pl
from jax.experimental.pallas import tpu as pltpu

_FWD_WEIGHT = 1.0


def _l1_kernel(x_ref, g_ref, m_ref, o_ref):
    d = jnp.abs(x_ref[...] - g_ref[...])
    mf = m_ref[...].astype(jnp.float32)
    r = d.shape[0] // 8
    m3 = mf.reshape(r, 4, 128)
    idx = jax.lax.broadcasted_iota(jnp.int32, (r, 8, 128), 1) // 2
    mex = jnp.take_along_axis(m3, idx, axis=1).reshape(r * 8, 128)
    s = jnp.sum(d * mex).reshape(1, 1)

    i = pl.program_id(0)

    @pl.when(i == 0)
    def _init():
        o_ref[...] = jnp.zeros((1, 1), jnp.float32)

    o_ref[...] += s


def kernel(uv_points, uv_gt, object_mask):
    B, H, W, C = uv_points.shape
    nrow = B * H * (W // 128) * C
    mrow = B * H * (W // 128)

    def as_rows(a):
        return (
            a.reshape(B, H, W // 128, 128, C)
            .transpose(0, 1, 2, 4, 3)
            .reshape(nrow, 128)
        )

    x = as_rows(uv_points)
    g = as_rows(uv_gt)
    m = object_mask.view(jnp.uint8)

    R = 8192
    n_steps = nrow // R
    out = pl.pallas_call(
        _l1_kernel,
        grid=(n_steps,),
        in_specs=[
            pl.BlockSpec((R, 128), lambda i: (i, 0)),
            pl.BlockSpec((R, 128), lambda i: (i, 0)),
            pl.BlockSpec((R // 4096, 512, 512), lambda i: (i, 0, 0)),
        ],
        out_specs=pl.BlockSpec((1, 1), lambda i: (0, 0)),
        out_shape=jax.ShapeDtypeStruct((1, 1), jnp.float32),
    )(x, g, m)

    uv_loss = out[0, 0] / float(B * H)
    return (_FWD_WEIGHT * uv_loss, uv_loss)
